# baseline (speedup 1.0000x reference)
import functools

import jax
import jax.numpy as jnp
from jax import lax
from jax.experimental import pallas as pl
from jax.experimental.pallas import tpu as pltpu
from jax.experimental.pallas import tpu_sc as plsc

N = 10000
N_PAD = 10240
E = 320000
E_P = 327680
HID = 128
XP = 16
CR = 30.0
W = 128
NCH = 2
EC = E_P // NCH
BE = 1280
BN = 1024
NS = 16
NC = 2
RSL = N_PAD // NS


@functools.cache
def _mesh():
    return plsc.VectorSubcoreMesh(core_axis_name="core", subcore_axis_name="subcore")


def _silu(x):
    return x * jax.nn.sigmoid(x)


def _sc_gather(width, untiled, dtype=jnp.float32):
    @functools.partial(
        pl.kernel,
        out_type=(
            jax.ShapeDtypeStruct((EC, width), dtype),
            jax.ShapeDtypeStruct((EC, width), dtype),
        ),
        mesh=_mesh(),
        compiler_params=pltpu.CompilerParams(use_tc_tiling_on_sc=not untiled),
    )
    def gather_kernel(t_hbm, ir_hbm, ic_hbm, or_hbm, oc_hbm):
        def body(ir_v, ic_v, or_v, oc_v):
            pltpu.sync_copy(t_hbm.at[ir_v.at[0]], or_v)
            pltpu.sync_copy(t_hbm.at[ic_v.at[0]], oc_v)

        pltpu.emit_pipeline(
            body,
            grid=(EC // W,),
            in_specs=[
                pl.BlockSpec((1, W), lambda i: (0, i)),
                pl.BlockSpec((1, W), lambda i: (0, i)),
            ],
            out_specs=[
                pl.BlockSpec((W, width), lambda i: (i, 0)),
                pl.BlockSpec((W, width), lambda i: (i, 0)),
            ],
            core_axis_name=("core", "subcore"),
            dimension_semantics=(pltpu.PARALLEL,),
        )(ir_hbm, ic_hbm, or_hbm, oc_hbm)

    return gather_kernel


def _sc_scatter(width, untiled):
    @functools.partial(
        pl.kernel,
        out_type=jax.ShapeDtypeStruct((NC, N_PAD, width), jnp.float32),
        mesh=_mesh(),
        scratch_types=[pltpu.VMEM_SHARED((N_PAD, width), jnp.float32)],
        compiler_params=pltpu.CompilerParams(use_tc_tiling_on_sc=not untiled),
    )
    def scatter_kernel(ir_hbm, m_hbm, prev_hbm, agg_hbm, agg_sh):
        c = lax.axis_index("core")
        s = lax.axis_index("subcore")
        sl = pl.ds(s * RSL, RSL)
        pltpu.sync_copy(prev_hbm.at[c, sl], agg_sh.at[sl])
        plsc.subcore_barrier()

        def body(ir_v, m_v):
            pltpu.sync_copy(m_v, agg_sh.at[ir_v.at[0]], add=True)

        pltpu.emit_pipeline(
            body,
            grid=(EC // W,),
            in_specs=[
                pl.BlockSpec((1, W), lambda i: (0, i)),
                pl.BlockSpec((W, width), lambda i: (i, 0)),
            ],
            out_specs=[],
            core_axis_name=("core", "subcore"),
            dimension_semantics=(pltpu.PARALLEL,),
        )(ir_hbm, m_hbm)

        plsc.subcore_barrier()
        pltpu.sync_copy(agg_sh.at[sl], agg_hbm.at[c, sl])

    return scatter_kernel


def _edge_body(hr_ref, hc_ref, xr_ref, xc_ref, wa_ref, wb_ref, wr_ref, b1_ref,
               w2_ref, b2_ref, watt_ref, batt_ref, wc1_ref, bc1_ref, wc2_ref,
               m_ref, tr_ref):
    hr = hr_ref[...]
    hc = hc_ref[...]
    d = xr_ref[...] - xc_ref[...]
    radial = jnp.sum(d * d, axis=1, keepdims=True)
    bf = jnp.bfloat16
    pre = (jnp.dot(hr.astype(bf), wa_ref[...].astype(bf),
                   preferred_element_type=jnp.float32)
           + jnp.dot(hc.astype(bf), wb_ref[...].astype(bf),
                     preferred_element_type=jnp.float32)
           + radial * wr_ref[...] + b1_ref[...])
    m1 = _silu(pre)
    m = _silu(jnp.dot(m1.astype(bf), w2_ref[...].astype(bf),
                      preferred_element_type=jnp.float32)
              + b2_ref[...])
    att = jax.nn.sigmoid(
        jnp.sum(m * watt_ref[...], axis=1, keepdims=True) + batt_ref[0, 0])
    m = m * att
    c1 = _silu(jnp.dot(m.astype(bf), wc1_ref[...].astype(bf),
                       preferred_element_type=jnp.float32)
               + bc1_ref[...])
    t = jnp.tanh(jnp.sum(c1 * wc2_ref[...], axis=1, keepdims=True)) * CR
    m_ref[...] = m
    tr_ref[...] = d * t


def _tc_edge(hr, hc, xr, xc, wa, wb, wr, b1, w2, b2, watt, batt, wc1, bc1, wc2):
    full = lambda shape: pl.BlockSpec(shape, lambda i: (0, 0))
    return pl.pallas_call(
        _edge_body,
        grid=(EC // BE,),
        in_specs=[
            pl.BlockSpec((BE, HID), lambda i: (i, 0)),
            pl.BlockSpec((BE, HID), lambda i: (i, 0)),
            pl.BlockSpec((BE, XP), lambda i: (i, 0)),
            pl.BlockSpec((BE, XP), lambda i: (i, 0)),
            full((HID, HID)), full((HID, HID)), full((1, HID)), full((1, HID)),
            full((HID, HID)), full((1, HID)), full((1, HID)), full((1, 1)),
            full((HID, HID)), full((1, HID)), full((1, HID)),
        ],
        out_specs=[
            pl.BlockSpec((BE, HID), lambda i: (i, 0)),
            pl.BlockSpec((BE, XP), lambda i: (i, 0)),
        ],
        out_shape=[
            jax.ShapeDtypeStruct((EC, HID), jnp.float32),
            jax.ShapeDtypeStruct((EC, XP), jnp.float32),
        ],
    )(hr, hc, xr, xc, wa, wb, wr, b1, w2, b2, watt, batt, wc1, bc1, wc2)


def _node_body(h_ref, x_ref, a0_ref, a1_ref, t0_ref, t1_ref, mask_ref,
               wna_ref, wnb_ref, bn1_ref, wn2_ref, bn2_ref, h_out, x_out):
    h = h_ref[...]
    agg = a0_ref[0] + a1_ref[0]
    u = _silu(jnp.dot(h, wna_ref[...], preferred_element_type=jnp.float32)
              + jnp.dot(agg, wnb_ref[...], preferred_element_type=jnp.float32)
              + bn1_ref[...])
    hn = (h + jnp.dot(u, wn2_ref[...], preferred_element_type=jnp.float32)
          + bn2_ref[...]) * mask_ref[...]
    h_out[...] = hn
    x_out[...] = x_ref[...] + t0_ref[0] + t1_ref[0]


def _tc_node(h, xpad, agg2, tra2, mask128, wna, wnb, bn1, wn2, bn2):
    full = lambda shape: pl.BlockSpec(shape, lambda i: (0, 0))
    return pl.pallas_call(
        _node_body,
        grid=(N_PAD // BN,),
        in_specs=[
            pl.BlockSpec((BN, HID), lambda i: (i, 0)),
            pl.BlockSpec((BN, XP), lambda i: (i, 0)),
            pl.BlockSpec((1, BN, HID), lambda i: (0, i, 0)),
            pl.BlockSpec((1, BN, HID), lambda i: (1, i, 0)),
            pl.BlockSpec((1, BN, XP), lambda i: (0, i, 0)),
            pl.BlockSpec((1, BN, XP), lambda i: (1, i, 0)),
            pl.BlockSpec((BN, HID), lambda i: (i, 0)),
            full((HID, HID)), full((HID, HID)), full((1, HID)),
            full((HID, HID)), full((1, HID)),
        ],
        out_specs=[
            pl.BlockSpec((BN, HID), lambda i: (i, 0)),
            pl.BlockSpec((BN, XP), lambda i: (i, 0)),
        ],
        out_shape=[
            jax.ShapeDtypeStruct((N_PAD, HID), jnp.float32),
            jax.ShapeDtypeStruct((N_PAD, XP), jnp.float32),
        ],
    )(h, xpad, agg2, agg2, tra2, tra2, mask128, wna, wnb, bn1, wn2, bn2)


def kernel(feature, vocab, size, pos, edge_index, mask, val, predict_idx, params):
    B = feature.shape[0]

    v_e = params["v_emb"][vocab]
    f_e = feature @ params["f1"]["w"] + params["f1"]["b"]
    f_e = _silu(f_e) @ params["f2"]["w"] + params["f2"]["b"]
    s_e = params["s_emb"][size]
    combined = jnp.concatenate([v_e, f_e, s_e], axis=2)
    combined = _silu(combined @ params["p1"]["w"] + params["p1"]["b"])
    combined = _silu(combined @ params["p2"]["w"] + params["p2"]["b"])
    combined = combined @ params["p3"]["w"] + params["p3"]["b"]
    h = (combined * mask).reshape(B * N, HID)
    h = jnp.pad(h, ((0, N_PAD - N), (0, 0)))

    xpad = jnp.pad(pos.reshape(B * N, 3), ((0, N_PAD - N), (0, XP - 3)))
    mask128 = jnp.pad(
        jnp.broadcast_to(mask.reshape(B * N, 1), (B * N, HID)),
        ((0, N_PAD - N), (0, 0)))

    dummy = jnp.full((1, E_P - E), N_PAD - 1, jnp.int32)
    row = jnp.concatenate(
        [edge_index[0].astype(jnp.int32).reshape(1, E), dummy], axis=1)
    col = jnp.concatenate(
        [edge_index[1].astype(jnp.int32).reshape(1, E), dummy], axis=1)

    z_hid = jnp.zeros((NC, N_PAD, HID), jnp.float32)
    z_xp = jnp.zeros((NC, N_PAD, XP), jnp.float32)

    row_ch = [row[:, k * EC:(k + 1) * EC] for k in range(NCH)]
    col_ch = [col[:, k * EC:(k + 1) * EC] for k in range(NCH)]

    for i in range(9):
        p = params["gcls"][i]
        r_ch, c_ch = (col_ch, row_ch) if 3 <= i < 6 else (row_ch, col_ch)
        w1 = p["edge1"]["w"]
        wa, wb = w1[:HID], w1[HID:2 * HID]
        wr = (w1[2 * HID] + w1[2 * HID + 1]).reshape(1, HID)
        b1 = p["edge1"]["b"].reshape(1, HID)
        w2, b2 = p["edge2"]["w"], p["edge2"]["b"].reshape(1, HID)
        watt = p["att"]["w"].reshape(1, HID)
        batt = p["att"]["b"].reshape(1, 1)
        wc1, bc1 = p["coord1"]["w"], p["coord1"]["b"].reshape(1, HID)
        wc2 = p["coord2"]["w"].reshape(1, HID)
        wn1 = p["node1"]["w"]
        wna, wnb = wn1[:HID], wn1[HID:]
        bn1 = p["node1"]["b"].reshape(1, HID)
        wn2, bn2 = p["node2"]["w"], p["node2"]["b"].reshape(1, HID)

        agg2, tra2 = z_hid, z_xp
        for k in range(NCH):
            r, c = r_ch[k], c_ch[k]
            hr, hc = _sc_gather(HID, untiled=False)(h, r, c)
            xr, xc = _sc_gather(XP, untiled=True)(xpad, r, c)
            m, tr = _tc_edge(hr, hc, xr, xc, wa, wb,
                             wr, b1, w2, b2, watt, batt, wc1, bc1, wc2)
            agg2 = _sc_scatter(HID, untiled=False)(r, m, agg2)
            tra2 = _sc_scatter(XP, untiled=True)(r, tr, tra2)
        h, xpad = _tc_node(h, xpad, agg2, tra2, mask128,
                           wna, wnb, bn1, wn2, bn2)

    h = h[:N].reshape(B, N, HID)
    hp = h[jnp.arange(B), predict_idx]
    o = jnp.concatenate([hp, val[:, None]], axis=1)
    o = _silu(o @ params["o1"]["w"] + params["o1"]["b"])
    return o @ params["o2"]["w"] + params["o2"]["b"]


# speedup vs baseline: 1.0877x; 1.0877x over previous
import functools

import jax
import jax.numpy as jnp
from jax import lax
from jax.experimental import pallas as pl
from jax.experimental.pallas import tpu as pltpu
from jax.experimental.pallas import tpu_sc as plsc

N = 10000
N_PAD = 10240
E = 320000
E_P = 327680
HID = 128
XP = 16
CR = 30.0
W = 128
NCH = 4
EC = E_P // NCH
BE = 1280
BN = 1024
NS = 16
NC = 2
RSL = N_PAD // NS


@functools.cache
def _mesh():
    return plsc.VectorSubcoreMesh(core_axis_name="core", subcore_axis_name="subcore")


def _silu(x):
    return x * jax.nn.sigmoid(x)


def _sc_gather(width, untiled, dtype=jnp.float32):
    @functools.partial(
        pl.kernel,
        out_type=(
            jax.ShapeDtypeStruct((EC, width), dtype),
            jax.ShapeDtypeStruct((EC, width), dtype),
        ),
        mesh=_mesh(),
        compiler_params=pltpu.CompilerParams(use_tc_tiling_on_sc=not untiled),
    )
    def gather_kernel(t_hbm, ir_hbm, ic_hbm, or_hbm, oc_hbm):
        def body(ir_v, ic_v, or_v, oc_v):
            pltpu.sync_copy(t_hbm.at[ir_v.at[0]], or_v)
            pltpu.sync_copy(t_hbm.at[ic_v.at[0]], oc_v)

        pltpu.emit_pipeline(
            body,
            grid=(EC // W,),
            in_specs=[
                pl.BlockSpec((1, W), lambda i: (0, i)),
                pl.BlockSpec((1, W), lambda i: (0, i)),
            ],
            out_specs=[
                pl.BlockSpec((W, width), lambda i: (i, 0)),
                pl.BlockSpec((W, width), lambda i: (i, 0)),
            ],
            core_axis_name=("core", "subcore"),
            dimension_semantics=(pltpu.PARALLEL,),
        )(ir_hbm, ic_hbm, or_hbm, oc_hbm)

    return gather_kernel


def _sc_scatter(width, untiled):
    @functools.partial(
        pl.kernel,
        out_type=jax.ShapeDtypeStruct((NC, N_PAD, width), jnp.float32),
        mesh=_mesh(),
        scratch_types=[pltpu.VMEM_SHARED((N_PAD, width), jnp.float32)],
        compiler_params=pltpu.CompilerParams(use_tc_tiling_on_sc=not untiled),
    )
    def scatter_kernel(ir_hbm, m_hbm, prev_hbm, agg_hbm, agg_sh):
        c = lax.axis_index("core")
        s = lax.axis_index("subcore")
        sl = pl.ds(s * RSL, RSL)
        pltpu.sync_copy(prev_hbm.at[c, sl], agg_sh.at[sl])
        plsc.subcore_barrier()

        def body(ir_v, m_v):
            pltpu.sync_copy(m_v, agg_sh.at[ir_v.at[0]], add=True)

        pltpu.emit_pipeline(
            body,
            grid=(EC // W,),
            in_specs=[
                pl.BlockSpec((1, W), lambda i: (0, i)),
                pl.BlockSpec((W, width), lambda i: (i, 0)),
            ],
            out_specs=[],
            core_axis_name=("core", "subcore"),
            dimension_semantics=(pltpu.PARALLEL,),
        )(ir_hbm, m_hbm)

        plsc.subcore_barrier()
        pltpu.sync_copy(agg_sh.at[sl], agg_hbm.at[c, sl])

    return scatter_kernel


def _edge_body(hr_ref, hc_ref, xr_ref, xc_ref, wa_ref, wb_ref, wr_ref, b1_ref,
               w2_ref, b2_ref, watt_ref, batt_ref, wc1_ref, bc1_ref, wc2_ref,
               m_ref, tr_ref):
    hr = hr_ref[...]
    hc = hc_ref[...]
    d = xr_ref[...] - xc_ref[...]
    radial = jnp.sum(d * d, axis=1, keepdims=True)
    bf = jnp.bfloat16
    pre = (jnp.dot(hr.astype(bf), wa_ref[...].astype(bf),
                   preferred_element_type=jnp.float32)
           + jnp.dot(hc.astype(bf), wb_ref[...].astype(bf),
                     preferred_element_type=jnp.float32)
           + radial * wr_ref[...] + b1_ref[...])
    m1 = _silu(pre)
    m = _silu(jnp.dot(m1.astype(bf), w2_ref[...].astype(bf),
                      preferred_element_type=jnp.float32)
              + b2_ref[...])
    att = jax.nn.sigmoid(
        jnp.sum(m * watt_ref[...], axis=1, keepdims=True) + batt_ref[0, 0])
    m = m * att
    c1 = _silu(jnp.dot(m.astype(bf), wc1_ref[...].astype(bf),
                       preferred_element_type=jnp.float32)
               + bc1_ref[...])
    t = jnp.tanh(jnp.sum(c1 * wc2_ref[...], axis=1, keepdims=True)) * CR
    m_ref[...] = m
    tr_ref[...] = d * t


def _tc_edge(hr, hc, xr, xc, wa, wb, wr, b1, w2, b2, watt, batt, wc1, bc1, wc2):
    full = lambda shape: pl.BlockSpec(shape, lambda i: (0, 0))
    return pl.pallas_call(
        _edge_body,
        grid=(EC // BE,),
        in_specs=[
            pl.BlockSpec((BE, HID), lambda i: (i, 0)),
            pl.BlockSpec((BE, HID), lambda i: (i, 0)),
            pl.BlockSpec((BE, XP), lambda i: (i, 0)),
            pl.BlockSpec((BE, XP), lambda i: (i, 0)),
            full((HID, HID)), full((HID, HID)), full((1, HID)), full((1, HID)),
            full((HID, HID)), full((1, HID)), full((1, HID)), full((1, 1)),
            full((HID, HID)), full((1, HID)), full((1, HID)),
        ],
        out_specs=[
            pl.BlockSpec((BE, HID), lambda i: (i, 0)),
            pl.BlockSpec((BE, XP), lambda i: (i, 0)),
        ],
        out_shape=[
            jax.ShapeDtypeStruct((EC, HID), jnp.float32),
            jax.ShapeDtypeStruct((EC, XP), jnp.float32),
        ],
    )(hr, hc, xr, xc, wa, wb, wr, b1, w2, b2, watt, batt, wc1, bc1, wc2)


def _node_body(h_ref, x_ref, a0_ref, a1_ref, t0_ref, t1_ref, mask_ref,
               wna_ref, wnb_ref, bn1_ref, wn2_ref, bn2_ref, h_out, x_out):
    h = h_ref[...]
    agg = a0_ref[0] + a1_ref[0]
    u = _silu(jnp.dot(h, wna_ref[...], preferred_element_type=jnp.float32)
              + jnp.dot(agg, wnb_ref[...], preferred_element_type=jnp.float32)
              + bn1_ref[...])
    hn = (h + jnp.dot(u, wn2_ref[...], preferred_element_type=jnp.float32)
          + bn2_ref[...]) * mask_ref[...]
    h_out[...] = hn
    x_out[...] = x_ref[...] + t0_ref[0] + t1_ref[0]


def _tc_node(h, xpad, agg2, tra2, mask128, wna, wnb, bn1, wn2, bn2):
    full = lambda shape: pl.BlockSpec(shape, lambda i: (0, 0))
    return pl.pallas_call(
        _node_body,
        grid=(N_PAD // BN,),
        in_specs=[
            pl.BlockSpec((BN, HID), lambda i: (i, 0)),
            pl.BlockSpec((BN, XP), lambda i: (i, 0)),
            pl.BlockSpec((1, BN, HID), lambda i: (0, i, 0)),
            pl.BlockSpec((1, BN, HID), lambda i: (1, i, 0)),
            pl.BlockSpec((1, BN, XP), lambda i: (0, i, 0)),
            pl.BlockSpec((1, BN, XP), lambda i: (1, i, 0)),
            pl.BlockSpec((BN, HID), lambda i: (i, 0)),
            full((HID, HID)), full((HID, HID)), full((1, HID)),
            full((HID, HID)), full((1, HID)),
        ],
        out_specs=[
            pl.BlockSpec((BN, HID), lambda i: (i, 0)),
            pl.BlockSpec((BN, XP), lambda i: (i, 0)),
        ],
        out_shape=[
            jax.ShapeDtypeStruct((N_PAD, HID), jnp.float32),
            jax.ShapeDtypeStruct((N_PAD, XP), jnp.float32),
        ],
    )(h, xpad, agg2, agg2, tra2, tra2, mask128, wna, wnb, bn1, wn2, bn2)


def kernel(feature, vocab, size, pos, edge_index, mask, val, predict_idx, params):
    B = feature.shape[0]

    v_e = params["v_emb"][vocab]
    f_e = feature @ params["f1"]["w"] + params["f1"]["b"]
    f_e = _silu(f_e) @ params["f2"]["w"] + params["f2"]["b"]
    s_e = params["s_emb"][size]
    combined = jnp.concatenate([v_e, f_e, s_e], axis=2)
    combined = _silu(combined @ params["p1"]["w"] + params["p1"]["b"])
    combined = _silu(combined @ params["p2"]["w"] + params["p2"]["b"])
    combined = combined @ params["p3"]["w"] + params["p3"]["b"]
    h = (combined * mask).reshape(B * N, HID)
    h = jnp.pad(h, ((0, N_PAD - N), (0, 0)))

    xpad = jnp.pad(pos.reshape(B * N, 3), ((0, N_PAD - N), (0, XP - 3)))
    mask128 = jnp.pad(
        jnp.broadcast_to(mask.reshape(B * N, 1), (B * N, HID)),
        ((0, N_PAD - N), (0, 0)))

    dummy = jnp.full((1, E_P - E), N_PAD - 1, jnp.int32)
    row = jnp.concatenate(
        [edge_index[0].astype(jnp.int32).reshape(1, E), dummy], axis=1)
    col = jnp.concatenate(
        [edge_index[1].astype(jnp.int32).reshape(1, E), dummy], axis=1)

    z_hid = jnp.zeros((NC, N_PAD, HID), jnp.float32)
    z_xp = jnp.zeros((NC, N_PAD, XP), jnp.float32)

    row_ch = [row[:, k * EC:(k + 1) * EC] for k in range(NCH)]
    col_ch = [col[:, k * EC:(k + 1) * EC] for k in range(NCH)]

    for i in range(9):
        p = params["gcls"][i]
        r_ch, c_ch = (col_ch, row_ch) if 3 <= i < 6 else (row_ch, col_ch)
        w1 = p["edge1"]["w"]
        wa, wb = w1[:HID], w1[HID:2 * HID]
        wr = (w1[2 * HID] + w1[2 * HID + 1]).reshape(1, HID)
        b1 = p["edge1"]["b"].reshape(1, HID)
        w2, b2 = p["edge2"]["w"], p["edge2"]["b"].reshape(1, HID)
        watt = p["att"]["w"].reshape(1, HID)
        batt = p["att"]["b"].reshape(1, 1)
        wc1, bc1 = p["coord1"]["w"], p["coord1"]["b"].reshape(1, HID)
        wc2 = p["coord2"]["w"].reshape(1, HID)
        wn1 = p["node1"]["w"]
        wna, wnb = wn1[:HID], wn1[HID:]
        bn1 = p["node1"]["b"].reshape(1, HID)
        wn2, bn2 = p["node2"]["w"], p["node2"]["b"].reshape(1, HID)

        agg2, tra2 = z_hid, z_xp
        for k in range(NCH):
            r, c = r_ch[k], c_ch[k]
            hr, hc = _sc_gather(HID, untiled=False)(h, r, c)
            xr, xc = _sc_gather(XP, untiled=True)(xpad, r, c)
            m, tr = _tc_edge(hr, hc, xr, xc, wa, wb,
                             wr, b1, w2, b2, watt, batt, wc1, bc1, wc2)
            agg2 = _sc_scatter(HID, untiled=False)(r, m, agg2)
            tra2 = _sc_scatter(XP, untiled=True)(r, tr, tra2)
        h, xpad = _tc_node(h, xpad, agg2, tra2, mask128,
                           wna, wnb, bn1, wn2, bn2)

    h = h[:N].reshape(B, N, HID)
    hp = h[jnp.arange(B), predict_idx]
    o = jnp.concatenate([hp, val[:, None]], axis=1)
    o = _silu(o @ params["o1"]["w"] + params["o1"]["b"])
    return o @ params["o2"]["w"] + params["o2"]["b"]
